# baseline (speedup 1.0000x reference)
import functools

import jax
import jax.numpy as jnp
from jax import lax
from jax.experimental import pallas as pl
from jax.experimental.pallas import tpu as pltpu
from jax.experimental.pallas import tpu_sc as plsc

_B, _N, _S, _D1, _D2 = 8, 4096, 1024, 128, 256
_C0, _C1 = 256, 256
_TN = 4096
_NT = _N // _TN

_NC, _NS, _L = 2, 16, 16
_NW = _NC * _NS
_ROWS = _B * _N
_RPW = _ROWS // _NW
_CH = 64
_NCH = _RPW // _CH


def _stage_knn(x1_ref, x2t_ref, i_ref, w_ref):
    b = pl.program_id(0)
    x1 = x1_ref[0]
    x2t = x2t_ref[0]
    n1 = jnp.sum(x1 * x1, axis=1, keepdims=True)
    n2 = jnp.sum(x2t * x2t, axis=0, keepdims=True)
    dot = jax.lax.dot(x1, x2t, preferred_element_type=jnp.float32)
    d = n1 + n2 - 2.0 * dot

    m0 = jnp.min(d, axis=1, keepdims=True)
    m1 = jnp.min(jnp.where(d > m0, d, jnp.inf), axis=1, keepdims=True)
    m2 = jnp.min(jnp.where(d > m1, d, jnp.inf), axis=1, keepdims=True)

    iota = jax.lax.broadcasted_iota(jnp.int32, d.shape, 1)
    base = b * _S
    i0 = jnp.min(jnp.where(d == m0, iota, _S), axis=1, keepdims=True)
    i1 = jnp.min(jnp.where(d == m1, iota, _S), axis=1, keepdims=True)
    i2 = jnp.min(jnp.where(d == m2, iota, _S), axis=1, keepdims=True)
    i_ref[0, :, 0:1] = i0 + base
    i_ref[0, :, 1:2] = i1 + base
    i_ref[0, :, 2:3] = i2 + base

    r0 = 1.0 / (m0 + 1e-8)
    r1 = 1.0 / (m1 + 1e-8)
    r2 = 1.0 / (m2 + 1e-8)
    rs = r0 + r1 + r2
    w_ref[0, :, 0:1] = r0 / rs
    w_ref[0, :, 1:2] = r1 / rs
    w_ref[0, :, 2:3] = r2 / rs


def _sc_interp(t_hbm, i0_hbm, i1_hbm, i2_hbm, w0_hbm, w1_hbm, w2_hbm,
               out_hbm, i0_v, i1_v, i2_v, w0_v, w1_v, w2_v,
               r0_v, r1_v, r2_v, o_v, sem):
    wid = lax.axis_index("s") * _NC + lax.axis_index("c")
    base = wid * _RPW

    def chunk(c, carry):
        off = base + c * _CH
        pltpu.sync_copy(i0_hbm.at[pl.ds(off, _CH)], i0_v)
        pltpu.sync_copy(i1_hbm.at[pl.ds(off, _CH)], i1_v)
        pltpu.sync_copy(i2_hbm.at[pl.ds(off, _CH)], i2_v)
        pltpu.sync_copy(w0_hbm.at[pl.ds(off, _CH)], w0_v.at[pl.ds(0, _CH)])
        pltpu.sync_copy(w1_hbm.at[pl.ds(off, _CH)], w1_v.at[pl.ds(0, _CH)])
        pltpu.sync_copy(w2_hbm.at[pl.ds(off, _CH)], w2_v.at[pl.ds(0, _CH)])
        cp0 = pltpu.async_copy(t_hbm.at[i0_v], r0_v, sem)
        cp1 = pltpu.async_copy(t_hbm.at[i1_v], r1_v, sem)
        cp2 = pltpu.async_copy(t_hbm.at[i2_v], r2_v, sem)
        cp0.wait()
        cp1.wait()
        cp2.wait()

        def row(r, rcarry):
            a0 = w0_v[pl.ds(r, _L)][0]
            a1 = w1_v[pl.ds(r, _L)][0]
            a2 = w2_v[pl.ds(r, _L)][0]
            for cc in range(_D2 // _L):
                sl = pl.ds(cc * _L, _L)
                o_v[r, sl] = (r0_v[r, sl] * a0 + r1_v[r, sl] * a1
                              + r2_v[r, sl] * a2)
            return rcarry

        lax.fori_loop(0, _CH, row, 0)
        pltpu.sync_copy(o_v, out_hbm.at[pl.ds(off, _CH)])
        return carry

    lax.fori_loop(0, _NCH, chunk, 0)


def _stage_mlp0(p1_ref, x_ref, w0_ref, b0_ref, y0_ref, s0_ref, q0_ref):
    first = pl.program_id(0) == 0

    @pl.when(first)
    def _():
        s0_ref[...] = jnp.zeros_like(s0_ref)
        q0_ref[...] = jnp.zeros_like(q0_ref)

    x = x_ref[0]
    p1 = p1_ref[0]
    w0a = w0_ref[:, :_D1]
    w0b = w0_ref[:, _D1:]
    y = (jax.lax.dot(w0a, p1, preferred_element_type=jnp.float32)
         + jax.lax.dot_general(w0b, x, (((1,), (1,)), ((), ())),
                               preferred_element_type=jnp.float32)
         + b0_ref[...])
    y0_ref[0] = y
    s0_ref[...] += jnp.sum(y, axis=1, keepdims=True)
    q0_ref[...] += jnp.sum(y * y, axis=1, keepdims=True)


def _stage_mlp1(y0_ref, sc_ref, sh_ref, w1_ref, b1_ref, y1_ref, s1_ref, q1_ref):
    first = pl.program_id(0) == 0

    @pl.when(first)
    def _():
        s1_ref[...] = jnp.zeros_like(s1_ref)
        q1_ref[...] = jnp.zeros_like(q1_ref)

    h = jnp.maximum(y0_ref[0] * sc_ref[...] + sh_ref[...], 0.0)
    y = (jnp.dot(w1_ref[...], h, preferred_element_type=jnp.float32)
         + b1_ref[...])
    y1_ref[0] = y
    s1_ref[...] += jnp.sum(y, axis=1, keepdims=True)
    q1_ref[...] += jnp.sum(y * y, axis=1, keepdims=True)


def _stage_out(y1_ref, sc_ref, sh_ref, o_ref):
    o_ref[0] = jnp.maximum(y1_ref[0] * sc_ref[...] + sh_ref[...], 0.0)


def kernel(xyz1, xyz2, points1, points2, W0, b0, gamma0, beta0,
           W1, b1, gamma1, beta1):
    xyz2t = jnp.transpose(xyz2, (0, 2, 1))
    col = lambda v: v.reshape(-1, 1)

    idxs, ws = pl.pallas_call(
        _stage_knn,
        grid=(_B,),
        in_specs=[
            pl.BlockSpec((1, _TN, 3), lambda b: (b, 0, 0)),
            pl.BlockSpec((1, 3, _S), lambda b: (b, 0, 0)),
        ],
        out_specs=[
            pl.BlockSpec((1, _TN, 3), lambda b: (b, 0, 0)),
            pl.BlockSpec((1, _TN, 3), lambda b: (b, 0, 0)),
        ],
        out_shape=[
            jax.ShapeDtypeStruct((_B, _N, 3), jnp.int32),
            jax.ShapeDtypeStruct((_B, _N, 3), jnp.float32),
        ],
    )(xyz1, xyz2t)

    table = jnp.transpose(points2, (0, 2, 1)).reshape(_B * _S, _D2)
    gi = [idxs[..., k].reshape(_ROWS) for k in range(3)]
    gw = [ws[..., k].reshape(_ROWS) for k in range(3)]

    sc_call = functools.partial(
        pl.kernel,
        out_type=jax.ShapeDtypeStruct((_ROWS, _D2), jnp.float32),
        mesh=plsc.VectorSubcoreMesh(core_axis_name="c", subcore_axis_name="s"),
        scratch_types=[
            pltpu.VMEM((_CH,), jnp.int32),
            pltpu.VMEM((_CH,), jnp.int32),
            pltpu.VMEM((_CH,), jnp.int32),
            pltpu.VMEM((_CH + _L,), jnp.float32),
            pltpu.VMEM((_CH + _L,), jnp.float32),
            pltpu.VMEM((_CH + _L,), jnp.float32),
            pltpu.VMEM((_CH, _D2), jnp.float32),
            pltpu.VMEM((_CH, _D2), jnp.float32),
            pltpu.VMEM((_CH, _D2), jnp.float32),
            pltpu.VMEM((_CH, _D2), jnp.float32),
            pltpu.SemaphoreType.DMA,
        ],
    )(_sc_interp)
    interp = sc_call(table, gi[0], gi[1], gi[2], gw[0], gw[1], gw[2])
    interp3 = interp.reshape(_B, _N, _D2)

    y0, s0, q0 = pl.pallas_call(
        _stage_mlp0,
        grid=(_B,),
        in_specs=[
            pl.BlockSpec((1, _D1, _TN), lambda b: (b, 0, 0)),
            pl.BlockSpec((1, _TN, _D2), lambda b: (b, 0, 0)),
            pl.BlockSpec((_C0, _D1 + _D2), lambda b: (0, 0)),
            pl.BlockSpec((_C0, 1), lambda b: (0, 0)),
        ],
        out_specs=[
            pl.BlockSpec((1, _C0, _TN), lambda b: (b, 0, 0)),
            pl.BlockSpec((_C0, 1), lambda b: (0, 0)),
            pl.BlockSpec((_C0, 1), lambda b: (0, 0)),
        ],
        out_shape=[
            jax.ShapeDtypeStruct((_B, _C0, _N), jnp.float32),
            jax.ShapeDtypeStruct((_C0, 1), jnp.float32),
            jax.ShapeDtypeStruct((_C0, 1), jnp.float32),
        ],
    )(points1, interp3, W0, col(b0))

    cnt = float(_B * _N)
    mean0 = s0 / cnt
    var0 = q0 / cnt - mean0 * mean0
    sc0 = col(gamma0) / jnp.sqrt(var0 + 1e-5)
    sh0 = col(beta0) - mean0 * sc0

    y1, s1, q1 = pl.pallas_call(
        _stage_mlp1,
        grid=(_B,),
        in_specs=[
            pl.BlockSpec((1, _C0, _TN), lambda b: (b, 0, 0)),
            pl.BlockSpec((_C0, 1), lambda b: (0, 0)),
            pl.BlockSpec((_C0, 1), lambda b: (0, 0)),
            pl.BlockSpec((_C1, _C0), lambda b: (0, 0)),
            pl.BlockSpec((_C1, 1), lambda b: (0, 0)),
        ],
        out_specs=[
            pl.BlockSpec((1, _C1, _TN), lambda b: (b, 0, 0)),
            pl.BlockSpec((_C1, 1), lambda b: (0, 0)),
            pl.BlockSpec((_C1, 1), lambda b: (0, 0)),
        ],
        out_shape=[
            jax.ShapeDtypeStruct((_B, _C1, _N), jnp.float32),
            jax.ShapeDtypeStruct((_C1, 1), jnp.float32),
            jax.ShapeDtypeStruct((_C1, 1), jnp.float32),
        ],
    )(y0, sc0, sh0, W1, col(b1))

    mean1 = s1 / cnt
    var1 = q1 / cnt - mean1 * mean1
    sc1 = col(gamma1) / jnp.sqrt(var1 + 1e-5)
    sh1 = col(beta1) - mean1 * sc1

    out = pl.pallas_call(
        _stage_out,
        grid=(_B,),
        in_specs=[
            pl.BlockSpec((1, _C1, _N), lambda b: (b, 0, 0)),
            pl.BlockSpec((_C1, 1), lambda b: (0, 0)),
            pl.BlockSpec((_C1, 1), lambda b: (0, 0)),
        ],
        out_specs=pl.BlockSpec((1, _C1, _N), lambda b: (b, 0, 0)),
        out_shape=jax.ShapeDtypeStruct((_B, _C1, _N), jnp.float32),
    )(y1, sc1, sh1)
    return out


# speedup vs baseline: 1.2007x; 1.2007x over previous
import functools

import jax
import jax.numpy as jnp
from jax import lax
from jax.experimental import pallas as pl
from jax.experimental.pallas import tpu as pltpu
from jax.experimental.pallas import tpu_sc as plsc

_B, _N, _S, _D1, _D2 = 8, 4096, 1024, 128, 256
_C0, _C1 = 256, 256
_TN = 4096
_NT = _N // _TN

_NC, _NS, _L = 2, 16, 16
_NW = _NC * _NS
_ROWS = _B * _N
_RPW = _ROWS // _NW
_CH = 64
_NCH = _RPW // _CH


def _stage_knn(x1_ref, x2t_ref, i_ref, w_ref):
    b = pl.program_id(0)
    x1 = x1_ref[0]
    x2t = x2t_ref[0]
    n1 = jnp.sum(x1 * x1, axis=1, keepdims=True)
    n2 = jnp.sum(x2t * x2t, axis=0, keepdims=True)
    dot = jax.lax.dot(x1, x2t, preferred_element_type=jnp.float32)
    d = n1 + n2 - 2.0 * dot

    m0 = jnp.min(d, axis=1, keepdims=True)
    m1 = jnp.min(jnp.where(d > m0, d, jnp.inf), axis=1, keepdims=True)
    m2 = jnp.min(jnp.where(d > m1, d, jnp.inf), axis=1, keepdims=True)

    iota = jax.lax.broadcasted_iota(jnp.int32, d.shape, 1)
    base = b * _S
    i0 = jnp.min(jnp.where(d == m0, iota, _S), axis=1, keepdims=True)
    i1 = jnp.min(jnp.where(d == m1, iota, _S), axis=1, keepdims=True)
    i2 = jnp.min(jnp.where(d == m2, iota, _S), axis=1, keepdims=True)
    i_ref[0, :, 0:1] = i0 + base
    i_ref[0, :, 1:2] = i1 + base
    i_ref[0, :, 2:3] = i2 + base

    r0 = 1.0 / (m0 + 1e-8)
    r1 = 1.0 / (m1 + 1e-8)
    r2 = 1.0 / (m2 + 1e-8)
    rs = r0 + r1 + r2
    w_ref[0, :, 0:1] = r0 / rs
    w_ref[0, :, 1:2] = r1 / rs
    w_ref[0, :, 2:3] = r2 / rs


def _sc_interp(t_hbm, i0_hbm, i1_hbm, i2_hbm, w0_hbm, w1_hbm, w2_hbm,
               out_hbm, i0_v, i1_v, i2_v, w0_v, w1_v, w2_v,
               r00, r01, r02, r10, r11, r12, o_v, sem0, sem1):
    wid = lax.axis_index("s") * _NC + lax.axis_index("c")
    base = wid * _RPW

    pltpu.sync_copy(i0_hbm.at[pl.ds(base, _RPW)], i0_v)
    pltpu.sync_copy(i1_hbm.at[pl.ds(base, _RPW)], i1_v)
    pltpu.sync_copy(i2_hbm.at[pl.ds(base, _RPW)], i2_v)
    pltpu.sync_copy(w0_hbm.at[pl.ds(base, _RPW)], w0_v.at[pl.ds(0, _RPW)])
    pltpu.sync_copy(w1_hbm.at[pl.ds(base, _RPW)], w1_v.at[pl.ds(0, _RPW)])
    pltpu.sync_copy(w2_hbm.at[pl.ds(base, _RPW)], w2_v.at[pl.ds(0, _RPW)])

    bufs = ((r00, r01, r02, sem0), (r10, r11, r12, sem1))

    def issue(c, bi):
        r0, r1, r2, sem = bufs[bi]
        sl = pl.ds(c * _CH, _CH)
        pltpu.async_copy(t_hbm.at[i0_v.at[sl]], r0, sem)
        pltpu.async_copy(t_hbm.at[i1_v.at[sl]], r1, sem)
        pltpu.async_copy(t_hbm.at[i2_v.at[sl]], r2, sem)

    def drain(bi):
        r0, r1, r2, sem = bufs[bi]
        sl = pl.ds(0, _CH)
        pltpu.make_async_copy(t_hbm.at[i0_v.at[sl]], r0, sem).wait()
        pltpu.make_async_copy(t_hbm.at[i1_v.at[sl]], r1, sem).wait()
        pltpu.make_async_copy(t_hbm.at[i2_v.at[sl]], r2, sem).wait()

    issue(0, 0)

    def pair(g, carry):
        for b in range(2):
            c = g * 2 + b

            @pl.when(c + 1 < _NCH)
            def _():
                issue(c + 1, (b + 1) % 2)

            drain(b)
            r0, r1, r2, _sem = bufs[b]

            def row(r, rcarry):
                a0 = w0_v[pl.ds(c * _CH + r, _L)][0]
                a1 = w1_v[pl.ds(c * _CH + r, _L)][0]
                a2 = w2_v[pl.ds(c * _CH + r, _L)][0]
                for cc in range(_D2 // _L):
                    sl2 = pl.ds(cc * _L, _L)
                    o_v[r, sl2] = (r0[r, sl2] * a0 + r1[r, sl2] * a1
                                   + r2[r, sl2] * a2)
                return rcarry

            lax.fori_loop(0, _CH, row, 0)
            pltpu.sync_copy(o_v, out_hbm.at[pl.ds(base + c * _CH, _CH)])
        return carry

    lax.fori_loop(0, _NCH // 2, pair, 0)


def _stage_mlp0(p1_ref, x_ref, w0_ref, b0_ref, y0_ref, s0_ref, q0_ref):
    first = pl.program_id(0) == 0

    @pl.when(first)
    def _():
        s0_ref[...] = jnp.zeros_like(s0_ref)
        q0_ref[...] = jnp.zeros_like(q0_ref)

    x = x_ref[0]
    p1 = p1_ref[0]
    w0a = w0_ref[:, :_D1]
    w0b = w0_ref[:, _D1:]
    y = (jax.lax.dot(w0a, p1, preferred_element_type=jnp.float32)
         + jax.lax.dot_general(w0b, x, (((1,), (1,)), ((), ())),
                               preferred_element_type=jnp.float32)
         + b0_ref[...])
    y0_ref[0] = y
    s0_ref[...] += jnp.sum(y, axis=1, keepdims=True)
    q0_ref[...] += jnp.sum(y * y, axis=1, keepdims=True)


def _stage_mlp1(y0_ref, sc_ref, sh_ref, w1_ref, b1_ref, y1_ref, s1_ref, q1_ref):
    first = pl.program_id(0) == 0

    @pl.when(first)
    def _():
        s1_ref[...] = jnp.zeros_like(s1_ref)
        q1_ref[...] = jnp.zeros_like(q1_ref)

    h = jnp.maximum(y0_ref[0] * sc_ref[...] + sh_ref[...], 0.0)
    y = (jnp.dot(w1_ref[...], h, preferred_element_type=jnp.float32)
         + b1_ref[...])
    y1_ref[0] = y
    s1_ref[...] += jnp.sum(y, axis=1, keepdims=True)
    q1_ref[...] += jnp.sum(y * y, axis=1, keepdims=True)


def _stage_out(y1_ref, sc_ref, sh_ref, o_ref):
    o_ref[0] = jnp.maximum(y1_ref[0] * sc_ref[...] + sh_ref[...], 0.0)


def kernel(xyz1, xyz2, points1, points2, W0, b0, gamma0, beta0,
           W1, b1, gamma1, beta1):
    xyz2t = jnp.transpose(xyz2, (0, 2, 1))
    col = lambda v: v.reshape(-1, 1)

    idxs, ws = pl.pallas_call(
        _stage_knn,
        grid=(_B,),
        in_specs=[
            pl.BlockSpec((1, _TN, 3), lambda b: (b, 0, 0)),
            pl.BlockSpec((1, 3, _S), lambda b: (b, 0, 0)),
        ],
        out_specs=[
            pl.BlockSpec((1, _TN, 3), lambda b: (b, 0, 0)),
            pl.BlockSpec((1, _TN, 3), lambda b: (b, 0, 0)),
        ],
        out_shape=[
            jax.ShapeDtypeStruct((_B, _N, 3), jnp.int32),
            jax.ShapeDtypeStruct((_B, _N, 3), jnp.float32),
        ],
    )(xyz1, xyz2t)

    table = jnp.transpose(points2, (0, 2, 1)).reshape(_B * _S, _D2)
    gi = [idxs[..., k].reshape(_ROWS) for k in range(3)]
    gw = [ws[..., k].reshape(_ROWS) for k in range(3)]

    sc_call = functools.partial(
        pl.kernel,
        out_type=jax.ShapeDtypeStruct((_ROWS, _D2), jnp.float32),
        mesh=plsc.VectorSubcoreMesh(core_axis_name="c", subcore_axis_name="s"),
        scratch_types=[
            pltpu.VMEM((_RPW,), jnp.int32),
            pltpu.VMEM((_RPW,), jnp.int32),
            pltpu.VMEM((_RPW,), jnp.int32),
            pltpu.VMEM((_RPW + _L,), jnp.float32),
            pltpu.VMEM((_RPW + _L,), jnp.float32),
            pltpu.VMEM((_RPW + _L,), jnp.float32),
            pltpu.VMEM((_CH, _D2), jnp.float32),
            pltpu.VMEM((_CH, _D2), jnp.float32),
            pltpu.VMEM((_CH, _D2), jnp.float32),
            pltpu.VMEM((_CH, _D2), jnp.float32),
            pltpu.VMEM((_CH, _D2), jnp.float32),
            pltpu.VMEM((_CH, _D2), jnp.float32),
            pltpu.VMEM((_CH, _D2), jnp.float32),
            pltpu.SemaphoreType.DMA,
            pltpu.SemaphoreType.DMA,
        ],
    )(_sc_interp)
    interp = sc_call(table, gi[0], gi[1], gi[2], gw[0], gw[1], gw[2])
    interp3 = interp.reshape(_B, _N, _D2)

    y0, s0, q0 = pl.pallas_call(
        _stage_mlp0,
        grid=(_B,),
        in_specs=[
            pl.BlockSpec((1, _D1, _TN), lambda b: (b, 0, 0)),
            pl.BlockSpec((1, _TN, _D2), lambda b: (b, 0, 0)),
            pl.BlockSpec((_C0, _D1 + _D2), lambda b: (0, 0)),
            pl.BlockSpec((_C0, 1), lambda b: (0, 0)),
        ],
        out_specs=[
            pl.BlockSpec((1, _C0, _TN), lambda b: (b, 0, 0)),
            pl.BlockSpec((_C0, 1), lambda b: (0, 0)),
            pl.BlockSpec((_C0, 1), lambda b: (0, 0)),
        ],
        out_shape=[
            jax.ShapeDtypeStruct((_B, _C0, _N), jnp.float32),
            jax.ShapeDtypeStruct((_C0, 1), jnp.float32),
            jax.ShapeDtypeStruct((_C0, 1), jnp.float32),
        ],
    )(points1, interp3, W0, col(b0))

    cnt = float(_B * _N)
    mean0 = s0 / cnt
    var0 = q0 / cnt - mean0 * mean0
    sc0 = col(gamma0) / jnp.sqrt(var0 + 1e-5)
    sh0 = col(beta0) - mean0 * sc0

    y1, s1, q1 = pl.pallas_call(
        _stage_mlp1,
        grid=(_B,),
        in_specs=[
            pl.BlockSpec((1, _C0, _TN), lambda b: (b, 0, 0)),
            pl.BlockSpec((_C0, 1), lambda b: (0, 0)),
            pl.BlockSpec((_C0, 1), lambda b: (0, 0)),
            pl.BlockSpec((_C1, _C0), lambda b: (0, 0)),
            pl.BlockSpec((_C1, 1), lambda b: (0, 0)),
        ],
        out_specs=[
            pl.BlockSpec((1, _C1, _TN), lambda b: (b, 0, 0)),
            pl.BlockSpec((_C1, 1), lambda b: (0, 0)),
            pl.BlockSpec((_C1, 1), lambda b: (0, 0)),
        ],
        out_shape=[
            jax.ShapeDtypeStruct((_B, _C1, _N), jnp.float32),
            jax.ShapeDtypeStruct((_C1, 1), jnp.float32),
            jax.ShapeDtypeStruct((_C1, 1), jnp.float32),
        ],
    )(y0, sc0, sh0, W1, col(b1))

    mean1 = s1 / cnt
    var1 = q1 / cnt - mean1 * mean1
    sc1 = col(gamma1) / jnp.sqrt(var1 + 1e-5)
    sh1 = col(beta1) - mean1 * sc1

    out = pl.pallas_call(
        _stage_out,
        grid=(_B,),
        in_specs=[
            pl.BlockSpec((1, _C1, _N), lambda b: (b, 0, 0)),
            pl.BlockSpec((_C1, 1), lambda b: (0, 0)),
            pl.BlockSpec((_C1, 1), lambda b: (0, 0)),
        ],
        out_specs=pl.BlockSpec((1, _C1, _N), lambda b: (b, 0, 0)),
        out_shape=jax.ShapeDtypeStruct((_B, _C1, _N), jnp.float32),
    )(y1, sc1, sh1)
    return out


# speedup vs baseline: 2.4011x; 1.9998x over previous
import jax
import jax.numpy as jnp
from jax.experimental import pallas as pl

_B, _N, _S, _D1, _D2 = 8, 4096, 1024, 128, 256
_C0, _C1 = 256, 256
_TN = 4096
_NT = _N // _TN


def _stage1(x1_ref, x2t_ref, p1_ref, p2_ref, w0_ref, b0_ref,
            y0_ref, s0_ref, q0_ref):
    first = (pl.program_id(0) == 0) & (pl.program_id(1) == 0)

    @pl.when(first)
    def _():
        s0_ref[...] = jnp.zeros_like(s0_ref)
        q0_ref[...] = jnp.zeros_like(q0_ref)

    x1 = x1_ref[0]
    x2t = x2t_ref[0]
    n1 = jnp.sum(x1 * x1, axis=1, keepdims=True)
    n2 = jnp.sum(x2t * x2t, axis=0, keepdims=True)
    dot = jax.lax.dot(x1, x2t, preferred_element_type=jnp.float32)
    d = n1 + n2 - 2.0 * dot

    m0 = jnp.min(d, axis=1, keepdims=True)
    m1 = jnp.min(jnp.where(d > m0, d, jnp.inf), axis=1, keepdims=True)
    m2 = jnp.min(jnp.where(d > m1, d, jnp.inf), axis=1, keepdims=True)

    r0 = 1.0 / (m0 + 1e-8)
    r1 = 1.0 / (m1 + 1e-8)
    r2 = 1.0 / (m2 + 1e-8)
    rs = r0 + r1 + r2
    a = jnp.where(d == m0, r0 / rs,
                  jnp.where(d == m1, r1 / rs,
                            jnp.where(d == m2, r2 / rs, 0.0)))

    p2 = p2_ref[0]
    interp = jax.lax.dot_general(a, p2, (((1,), (1,)), ((), ())),
                                 preferred_element_type=jnp.float32)

    p1 = p1_ref[0]
    w0a = w0_ref[:, :_D1]
    w0b = w0_ref[:, _D1:]
    y = (jax.lax.dot(w0a, p1, preferred_element_type=jnp.float32)
         + jax.lax.dot_general(w0b, interp, (((1,), (1,)), ((), ())),
                               preferred_element_type=jnp.float32)
         + b0_ref[...])
    y0_ref[0] = y
    s0_ref[...] += jnp.sum(y, axis=1, keepdims=True)
    q0_ref[...] += jnp.sum(y * y, axis=1, keepdims=True)


def _bn_coeffs(s_ref, q_ref, g_ref, be_ref):
    cnt = float(_B * _N)
    mean = s_ref[...] / cnt
    var = q_ref[...] / cnt - mean * mean
    sc = g_ref[...] / jnp.sqrt(var + 1e-5)
    sh = be_ref[...] - mean * sc
    return sc, sh


def _stage2(y0_ref, s0_ref, q0_ref, g0_ref, be0_ref, w1_ref, b1_ref,
            y1_ref, s1_ref, q1_ref):
    first = (pl.program_id(0) == 0) & (pl.program_id(1) == 0)

    @pl.when(first)
    def _():
        s1_ref[...] = jnp.zeros_like(s1_ref)
        q1_ref[...] = jnp.zeros_like(q1_ref)

    sc, sh = _bn_coeffs(s0_ref, q0_ref, g0_ref, be0_ref)
    h = jnp.maximum(y0_ref[0] * sc + sh, 0.0)
    y = (jnp.dot(w1_ref[...], h, preferred_element_type=jnp.float32)
         + b1_ref[...])
    y1_ref[0] = y
    s1_ref[...] += jnp.sum(y, axis=1, keepdims=True)
    q1_ref[...] += jnp.sum(y * y, axis=1, keepdims=True)


def _stage3(y1_ref, s1_ref, q1_ref, g1_ref, be1_ref, o_ref):
    sc, sh = _bn_coeffs(s1_ref, q1_ref, g1_ref, be1_ref)
    o_ref[0] = jnp.maximum(y1_ref[0] * sc + sh, 0.0)


def kernel(xyz1, xyz2, points1, points2, W0, b0, gamma0, beta0,
           W1, b1, gamma1, beta1):
    xyz2t = jnp.transpose(xyz2, (0, 2, 1))
    col = lambda v: v.reshape(-1, 1)

    y0, s0, q0 = pl.pallas_call(
        _stage1,
        grid=(_B, _NT),
        in_specs=[
            pl.BlockSpec((1, _TN, 3), lambda b, n: (b, n, 0)),
            pl.BlockSpec((1, 3, _S), lambda b, n: (b, 0, 0)),
            pl.BlockSpec((1, _D1, _TN), lambda b, n: (b, 0, n)),
            pl.BlockSpec((1, _D2, _S), lambda b, n: (b, 0, 0)),
            pl.BlockSpec((_C0, _D1 + _D2), lambda b, n: (0, 0)),
            pl.BlockSpec((_C0, 1), lambda b, n: (0, 0)),
        ],
        out_specs=[
            pl.BlockSpec((1, _C0, _TN), lambda b, n: (b, 0, n)),
            pl.BlockSpec((_C0, 1), lambda b, n: (0, 0)),
            pl.BlockSpec((_C0, 1), lambda b, n: (0, 0)),
        ],
        out_shape=[
            jax.ShapeDtypeStruct((_B, _C0, _N), jnp.float32),
            jax.ShapeDtypeStruct((_C0, 1), jnp.float32),
            jax.ShapeDtypeStruct((_C0, 1), jnp.float32),
        ],
    )(xyz1, xyz2t, points1, points2, W0, col(b0))

    y1, s1, q1 = pl.pallas_call(
        _stage2,
        grid=(_B, _NT),
        in_specs=[
            pl.BlockSpec((1, _C0, _TN), lambda b, n: (b, 0, n)),
            pl.BlockSpec((_C0, 1), lambda b, n: (0, 0)),
            pl.BlockSpec((_C0, 1), lambda b, n: (0, 0)),
            pl.BlockSpec((_C0, 1), lambda b, n: (0, 0)),
            pl.BlockSpec((_C0, 1), lambda b, n: (0, 0)),
            pl.BlockSpec((_C1, _C0), lambda b, n: (0, 0)),
            pl.BlockSpec((_C1, 1), lambda b, n: (0, 0)),
        ],
        out_specs=[
            pl.BlockSpec((1, _C1, _TN), lambda b, n: (b, 0, n)),
            pl.BlockSpec((_C1, 1), lambda b, n: (0, 0)),
            pl.BlockSpec((_C1, 1), lambda b, n: (0, 0)),
        ],
        out_shape=[
            jax.ShapeDtypeStruct((_B, _C1, _N), jnp.float32),
            jax.ShapeDtypeStruct((_C1, 1), jnp.float32),
            jax.ShapeDtypeStruct((_C1, 1), jnp.float32),
        ],
    )(y0, s0, q0, col(gamma0), col(beta0), W1, col(b1))

    out = pl.pallas_call(
        _stage3,
        grid=(_B,),
        in_specs=[
            pl.BlockSpec((1, _C1, _N), lambda b: (b, 0, 0)),
            pl.BlockSpec((_C1, 1), lambda b: (0, 0)),
            pl.BlockSpec((_C1, 1), lambda b: (0, 0)),
            pl.BlockSpec((_C1, 1), lambda b: (0, 0)),
            pl.BlockSpec((_C1, 1), lambda b: (0, 0)),
        ],
        out_specs=pl.BlockSpec((1, _C1, _N), lambda b: (b, 0, 0)),
        out_shape=jax.ShapeDtypeStruct((_B, _C1, _N), jnp.float32),
    )(y1, s1, q1, col(gamma1), col(beta1))
    return out
